# baseline (speedup 1.0000x reference)
import functools

import jax
import jax.numpy as jnp
from jax import lax
from jax.experimental import pallas as pl
from jax.experimental.pallas import tpu as pltpu
from jax.experimental.pallas import tpu_sc as plsc

_D = 16
_K = 16
_NL = 3
_H = 2 * _D + 1

_P = 256
_TA = 1024

_NC = 2
_NS = 16
_NW = _NC * _NS
_R = 128
_CR = 8


def _knn_body(t0_ref, t1_ref, x_ref, xb_ref, yt_ref, yb_ref, idx_ref, d2_ref,
              dscr):
    i = pl.program_id(0)
    t0 = t0_ref[i]
    t1 = t1_ref[i]
    xx = x_ref[:, 0:1]
    xy = x_ref[:, 1:2]
    xz = x_ref[:, 2:3]
    xb = xb_ref[:, 0:1]

    inf = jnp.float32(jnp.inf)
    big = jnp.int32(2**30)
    lane = lax.broadcasted_iota(jnp.int32, (_P, _TA), 1)

    nex = 4

    def topn_of_tile(dt, c0):
        out = []
        cur = dt
        for j in range(nex):
            m = jnp.min(cur, axis=1, keepdims=True)
            l = jnp.min(jnp.where(cur == m, lane, big), axis=1, keepdims=True)
            out.append((m, l + c0))
            if j + 1 < nex:
                cur = jnp.where(lane == l, inf, cur)
        return out

    def cmpx(a, b):
        c = a[0] <= b[0]
        lo = (jnp.where(c, a[0], b[0]), jnp.where(c, a[1], b[1]))
        hi = (jnp.where(c, b[0], a[0]), jnp.where(c, b[1], a[1]))
        return lo, hi

    def mergen(a, b):
        c = [None] * nex
        for j in range(nex):
            av, bv = a[j], b[nex - 1 - j]
            keep = av[0] <= bv[0]
            c[j] = (jnp.where(keep, av[0], bv[0]),
                    jnp.where(keep, av[1], bv[1]))
        c[0], c[2] = cmpx(c[0], c[2])
        c[1], c[3] = cmpx(c[1], c[3])
        c[0], c[1] = cmpx(c[0], c[1])
        c[2], c[3] = cmpx(c[2], c[3])
        return c

    def carry0():
        z = jnp.full((_P, 1), inf, jnp.float32)
        zi = jnp.zeros((_P, 1), jnp.int32)
        return [(z, zi)] * nex

    def pass0_body(t, carry):
        c0 = t * _TA
        dx = xx - yt_ref[0:1, pl.ds(c0, _TA)]
        dy = xy - yt_ref[1:2, pl.ds(c0, _TA)]
        dz = xz - yt_ref[2:3, pl.ds(c0, _TA)]
        dt = dx * dx + dy * dy + dz * dz
        dt = jnp.where(xb != yb_ref[0:1, pl.ds(c0, _TA)], inf, dt)
        dscr[:, pl.ds(c0, _TA)] = dt
        return mergen(carry, topn_of_tile(dt, c0))

    sel = lax.fori_loop(t0, t1, pass0_body, carry0())
    picks = list(sel)

    for _ in range(_K // nex - 1):
        vlast = picks[-1][0]

        def scan_body(t, carry, vlast=vlast):
            c0 = t * _TA
            dt = dscr[:, pl.ds(c0, _TA)]
            dt = jnp.where(dt <= vlast, inf, dt)
            return mergen(carry, topn_of_tile(dt, c0))

        sel = lax.fori_loop(t0, t1, scan_body, carry0())
        picks += list(sel)

    idx_ref[:, :] = jnp.concatenate([p[1] for p in picks], axis=1)
    d2_ref[:, :] = jnp.concatenate([p[0] for p in picks], axis=1)


def _knn_call(x, yt, xb2, yb2, t0, t1):
    n = x.shape[0]
    v = yt.shape[1]
    nb = n // _P
    return pl.pallas_call(
        _knn_body,
        grid=(nb,),
        in_specs=[
            pl.BlockSpec(memory_space=pltpu.SMEM),
            pl.BlockSpec(memory_space=pltpu.SMEM),
            pl.BlockSpec((_P, 3), lambda i: (i, 0)),
            pl.BlockSpec((_P, 1), lambda i: (i, 0)),
            pl.BlockSpec((3, v), lambda i: (0, 0)),
            pl.BlockSpec((1, v), lambda i: (0, 0)),
        ],
        out_specs=[
            pl.BlockSpec((_P, _K), lambda i: (i, 0)),
            pl.BlockSpec((_P, _K), lambda i: (i, 0)),
        ],
        out_shape=[
            jax.ShapeDtypeStruct((n, _K), jnp.int32),
            jax.ShapeDtypeStruct((n, _K), jnp.float32),
        ],
        scratch_shapes=[pltpu.VMEM((_P, v), jnp.float32)],
        compiler_params=pltpu.CompilerParams(
            dimension_semantics=("arbitrary",)),
    )(t0, t1, x, xb2, yt, yb2)


def _gather_body(tab_hbm, idx_hbm, out_hbm, idx_v, buf_v, sem):
    wid = lax.axis_index("s") * _NC + lax.axis_index("c")
    rows_per_w = idx_hbm.shape[0] // _NW
    base = wid * rows_per_w
    pltpu.sync_copy(idx_hbm.at[pl.ds(base, rows_per_w)], idx_v)

    def chunk(ci, carry):
        handles = []
        for j in range(_CR):
            r = ci * _CR + j
            h = pltpu.async_copy(
                tab_hbm.at[idx_v.at[r]],
                buf_v.at[pl.ds(j * _R, _R)],
                sem,
            )
            handles.append(h)
        for h in handles:
            h.wait()
        out_off = (base + ci * _CR) * _R
        pltpu.sync_copy(buf_v, out_hbm.at[pl.ds(out_off, _CR * _R)])
        return carry

    lax.fori_loop(0, rows_per_w // _CR, chunk, 0)


def _gather_call(table, idx_flat):
    b = idx_flat.shape[0]
    d = table.shape[1]
    idx2 = idx_flat.reshape(b // _R, _R)
    mesh = plsc.VectorSubcoreMesh(core_axis_name="c", subcore_axis_name="s")
    rows_per_w = idx2.shape[0] // _NW
    run = functools.partial(
        pl.kernel,
        mesh=mesh,
        out_type=jax.ShapeDtypeStruct((b, d), jnp.float32),
        scratch_types=[
            pltpu.VMEM((rows_per_w, _R), jnp.int32),
            pltpu.VMEM((_CR * _R, d), jnp.float32),
            pltpu.SemaphoreType.DMA,
        ],
        compiler_params=pltpu.CompilerParams(use_tc_tiling_on_sc=False),
    )(_gather_body)
    return run(table, idx2)


_PM = 512


def _mlp_body(af_ref, dt_ref, w1_ref, b1_ref, w2_ref, b2_ref, gw_ref, gb_ref,
              out_ref):
    pe = jnp.ones((_PM, _D), jnp.float32)
    for l in range(_NL):
        w1 = w1_ref[l]
        w1a = w1[0:_D, :]
        w1b = w1[_D:2 * _D, :]
        w1c = w1[2 * _D:2 * _D + 1, :]
        peh = jnp.dot(pe, w1a, preferred_element_type=jnp.float32) + b1_ref[l]
        hsum = jnp.zeros((_PM, _H), jnp.float32)
        for k in range(_K):
            af = af_ref[:, k * _D:(k + 1) * _D]
            dk = dt_ref[:, k:k + 1]
            hk = (peh + jnp.dot(af, w1b, preferred_element_type=jnp.float32)
                  + dk * w1c)
            hsum = hsum + jnp.where(hk >= 0, hk, 0.2 * hk)
        msg = (jnp.dot(hsum, w2_ref[l], preferred_element_type=jnp.float32)
               + jnp.float32(_K) * b2_ref[l])
        g1 = msg[:, 0:_D // 2]
        g2 = msg[:, _D // 2:_D]
        mu1 = jnp.mean(g1, axis=1, keepdims=True)
        mu2 = jnp.mean(g2, axis=1, keepdims=True)
        c1 = g1 - mu1
        c2 = g2 - mu2
        v1 = jnp.mean(c1 * c1, axis=1, keepdims=True)
        v2 = jnp.mean(c2 * c2, axis=1, keepdims=True)
        tn = jnp.concatenate(
            [c1 / jnp.sqrt(v1 + 1e-5), c2 / jnp.sqrt(v2 + 1e-5)], axis=1)
        tn = tn * gw_ref[l] + gb_ref[l]
        pe = pe + jnp.where(tn >= 0, tn, 0.2 * tn)
    out_ref[:, :] = pe


def _mlp_call(af2, d2, w1, b1, w2, b2, gw, gb):
    n = af2.shape[0]
    return pl.pallas_call(
        _mlp_body,
        grid=(n // _PM,),
        in_specs=[
            pl.BlockSpec((_PM, _K * _D), lambda i: (i, 0)),
            pl.BlockSpec((_PM, _K), lambda i: (i, 0)),
            pl.BlockSpec((_NL, _H, _H), lambda i: (0, 0, 0)),
            pl.BlockSpec((_NL, 1, _H), lambda i: (0, 0, 0)),
            pl.BlockSpec((_NL, _H, _D), lambda i: (0, 0, 0)),
            pl.BlockSpec((_NL, 1, _D), lambda i: (0, 0, 0)),
            pl.BlockSpec((_NL, 1, _D), lambda i: (0, 0, 0)),
            pl.BlockSpec((_NL, 1, _D), lambda i: (0, 0, 0)),
        ],
        out_specs=pl.BlockSpec((_PM, _D), lambda i: (i, 0)),
        out_shape=jax.ShapeDtypeStruct((n, _D), jnp.float32),
        compiler_params=pltpu.CompilerParams(
            dimension_semantics=("arbitrary",)),
    )(af2, d2, w1, b1, w2, b2, gw, gb)


def kernel(x, y, y_atomtypes, params, x_batch, y_batch):
    n = x.shape[0]

    xb_blk = x_batch.reshape(n // _P, _P)
    blo = xb_blk[:, 0]
    bhi = xb_blk[:, _P - 1]
    wlo = jnp.searchsorted(y_batch, blo, side="left").astype(jnp.int32)
    whi = jnp.searchsorted(y_batch, bhi, side="right").astype(jnp.int32)
    t0 = wlo // _TA
    t1 = (whi + _TA - 1) // _TA

    idx, d2 = _knn_call(
        x,
        y.T,
        x_batch.reshape(n, 1),
        y_batch.reshape(1, y.shape[0]),
        t0,
        t1,
    )

    af = _gather_call(y_atomtypes, idx.reshape(-1))
    af2 = af.reshape(n, _K * _D)

    w1 = jnp.stack(params["w1"])
    b1 = jnp.stack(params["b1"]).reshape(_NL, 1, _H)
    w2 = jnp.stack(params["w2"])
    b2 = jnp.stack(params["b2"]).reshape(_NL, 1, _D)
    gw = jnp.stack(params["gw"]).reshape(_NL, 1, _D)
    gb = jnp.stack(params["gb"]).reshape(_NL, 1, _D)

    return _mlp_call(af2, d2, w1, b1, w2, b2, gw, gb)


# speedup vs baseline: 1.0466x; 1.0466x over previous
import functools

import jax
import jax.numpy as jnp
from jax import lax
from jax.experimental import pallas as pl
from jax.experimental.pallas import tpu as pltpu
from jax.experimental.pallas import tpu_sc as plsc

_D = 16
_K = 16
_NL = 3
_H = 2 * _D + 1

_P = 256
_TA = 1024

_NC = 2
_NS = 16
_NW = _NC * _NS
_R = 128
_CR = 8


def _knn_body(t0_ref, t1_ref, x_ref, xb_ref, yt_ref, yb_ref, idx_ref, d2_ref,
              dscr):
    i = pl.program_id(0)
    t0 = t0_ref[i]
    t1 = t1_ref[i]
    xx = x_ref[:, 0:1]
    xy = x_ref[:, 1:2]
    xz = x_ref[:, 2:3]
    xb = xb_ref[:, 0:1]

    inf = jnp.float32(jnp.inf)
    big = jnp.int32(2**30)
    lane = lax.broadcasted_iota(jnp.int32, (_P, _TA), 1)

    nex = 2

    def topn_of_tile(dt, c0):
        out = []
        cur = dt
        for j in range(nex):
            m = jnp.min(cur, axis=1, keepdims=True)
            l = jnp.min(jnp.where(cur == m, lane, big), axis=1, keepdims=True)
            out.append((m, l + c0))
            if j + 1 < nex:
                cur = jnp.where(lane == l, inf, cur)
        return out

    def cmpx(a, b):
        c = a[0] <= b[0]
        lo = (jnp.where(c, a[0], b[0]), jnp.where(c, a[1], b[1]))
        hi = (jnp.where(c, b[0], a[0]), jnp.where(c, b[1], a[1]))
        return lo, hi

    def mergen(a, b):
        c = [None] * nex
        for j in range(nex):
            av, bv = a[j], b[nex - 1 - j]
            keep = av[0] <= bv[0]
            c[j] = (jnp.where(keep, av[0], bv[0]),
                    jnp.where(keep, av[1], bv[1]))
        if nex == 4:
            c[0], c[2] = cmpx(c[0], c[2])
            c[1], c[3] = cmpx(c[1], c[3])
            c[0], c[1] = cmpx(c[0], c[1])
            c[2], c[3] = cmpx(c[2], c[3])
        else:
            c[0], c[1] = cmpx(c[0], c[1])
        return c

    def carry0():
        z = jnp.full((_P, 1), inf, jnp.float32)
        zi = jnp.zeros((_P, 1), jnp.int32)
        return [(z, zi)] * nex

    def pass0_body(t, carry):
        c0 = t * _TA
        dx = xx - yt_ref[0:1, pl.ds(c0, _TA)]
        dy = xy - yt_ref[1:2, pl.ds(c0, _TA)]
        dz = xz - yt_ref[2:3, pl.ds(c0, _TA)]
        dt = dx * dx + dy * dy + dz * dz
        dt = jnp.where(xb != yb_ref[0:1, pl.ds(c0, _TA)], inf, dt)
        dscr[:, pl.ds(c0, _TA)] = dt
        return mergen(carry, topn_of_tile(dt, c0))

    sel = lax.fori_loop(t0, t1, pass0_body, carry0())
    picks = list(sel)

    for _ in range(_K // nex - 1):
        vlast = picks[-1][0]

        def scan_body(t, carry, vlast=vlast):
            c0 = t * _TA
            dt = dscr[:, pl.ds(c0, _TA)]
            dt = jnp.where(dt <= vlast, inf, dt)
            return mergen(carry, topn_of_tile(dt, c0))

        sel = lax.fori_loop(t0, t1, scan_body, carry0())
        picks += list(sel)

    idx_ref[:, :] = jnp.concatenate([p[1] for p in picks], axis=1)
    d2_ref[:, :] = jnp.concatenate([p[0] for p in picks], axis=1)


def _knn_call(x, yt, xb2, yb2, t0, t1):
    n = x.shape[0]
    v = yt.shape[1]
    nb = n // _P
    return pl.pallas_call(
        _knn_body,
        grid=(nb,),
        in_specs=[
            pl.BlockSpec(memory_space=pltpu.SMEM),
            pl.BlockSpec(memory_space=pltpu.SMEM),
            pl.BlockSpec((_P, 3), lambda i: (i, 0)),
            pl.BlockSpec((_P, 1), lambda i: (i, 0)),
            pl.BlockSpec((3, v), lambda i: (0, 0)),
            pl.BlockSpec((1, v), lambda i: (0, 0)),
        ],
        out_specs=[
            pl.BlockSpec((_P, _K), lambda i: (i, 0)),
            pl.BlockSpec((_P, _K), lambda i: (i, 0)),
        ],
        out_shape=[
            jax.ShapeDtypeStruct((n, _K), jnp.int32),
            jax.ShapeDtypeStruct((n, _K), jnp.float32),
        ],
        scratch_shapes=[pltpu.VMEM((_P, v), jnp.float32)],
        compiler_params=pltpu.CompilerParams(
            dimension_semantics=("arbitrary",)),
    )(t0, t1, x, xb2, yt, yb2)


def _gather_body(tab_hbm, idx_hbm, out_hbm, idx_v, buf_v, sem):
    wid = lax.axis_index("s") * _NC + lax.axis_index("c")
    rows_per_w = idx_hbm.shape[0] // _NW
    base = wid * rows_per_w
    pltpu.sync_copy(idx_hbm.at[pl.ds(base, rows_per_w)], idx_v)

    def chunk(ci, carry):
        handles = []
        for j in range(_CR):
            r = ci * _CR + j
            h = pltpu.async_copy(
                tab_hbm.at[idx_v.at[r]],
                buf_v.at[pl.ds(j * _R, _R)],
                sem,
            )
            handles.append(h)
        for h in handles:
            h.wait()
        out_off = (base + ci * _CR) * _R
        pltpu.sync_copy(buf_v, out_hbm.at[pl.ds(out_off, _CR * _R)])
        return carry

    lax.fori_loop(0, rows_per_w // _CR, chunk, 0)


def _gather_call(table, idx_flat):
    b = idx_flat.shape[0]
    d = table.shape[1]
    idx2 = idx_flat.reshape(b // _R, _R)
    mesh = plsc.VectorSubcoreMesh(core_axis_name="c", subcore_axis_name="s")
    rows_per_w = idx2.shape[0] // _NW
    run = functools.partial(
        pl.kernel,
        mesh=mesh,
        out_type=jax.ShapeDtypeStruct((b, d), jnp.float32),
        scratch_types=[
            pltpu.VMEM((rows_per_w, _R), jnp.int32),
            pltpu.VMEM((_CR * _R, d), jnp.float32),
            pltpu.SemaphoreType.DMA,
        ],
        compiler_params=pltpu.CompilerParams(use_tc_tiling_on_sc=False),
    )(_gather_body)
    return run(table, idx2)


_PM = 1024


def _mlp_body(af_ref, dt_ref, w1_ref, b1_ref, w2_ref, b2_ref, gw_ref, gb_ref,
              out_ref):
    pe = jnp.ones((_PM, _D), jnp.float32)
    for l in range(_NL):
        w1 = w1_ref[l]
        w1a = w1[0:_D, :]
        w1b = w1[_D:2 * _D, :]
        w1c = w1[2 * _D:2 * _D + 1, :]
        peh = jnp.dot(pe, w1a, preferred_element_type=jnp.float32) + b1_ref[l]
        hsum = jnp.zeros((_PM, _H), jnp.float32)
        for k in range(_K):
            af = af_ref[:, k * _D:(k + 1) * _D]
            dk = dt_ref[:, k:k + 1]
            hk = (peh + jnp.dot(af, w1b, preferred_element_type=jnp.float32)
                  + dk * w1c)
            hsum = hsum + jnp.where(hk >= 0, hk, 0.2 * hk)
        msg = (jnp.dot(hsum, w2_ref[l], preferred_element_type=jnp.float32)
               + jnp.float32(_K) * b2_ref[l])
        g1 = msg[:, 0:_D // 2]
        g2 = msg[:, _D // 2:_D]
        mu1 = jnp.mean(g1, axis=1, keepdims=True)
        mu2 = jnp.mean(g2, axis=1, keepdims=True)
        c1 = g1 - mu1
        c2 = g2 - mu2
        v1 = jnp.mean(c1 * c1, axis=1, keepdims=True)
        v2 = jnp.mean(c2 * c2, axis=1, keepdims=True)
        tn = jnp.concatenate(
            [c1 / jnp.sqrt(v1 + 1e-5), c2 / jnp.sqrt(v2 + 1e-5)], axis=1)
        tn = tn * gw_ref[l] + gb_ref[l]
        pe = pe + jnp.where(tn >= 0, tn, 0.2 * tn)
    out_ref[:, :] = pe


def _mlp_call(af2, d2, w1, b1, w2, b2, gw, gb):
    n = af2.shape[0]
    return pl.pallas_call(
        _mlp_body,
        grid=(n // _PM,),
        in_specs=[
            pl.BlockSpec((_PM, _K * _D), lambda i: (i, 0)),
            pl.BlockSpec((_PM, _K), lambda i: (i, 0)),
            pl.BlockSpec((_NL, _H, _H), lambda i: (0, 0, 0)),
            pl.BlockSpec((_NL, 1, _H), lambda i: (0, 0, 0)),
            pl.BlockSpec((_NL, _H, _D), lambda i: (0, 0, 0)),
            pl.BlockSpec((_NL, 1, _D), lambda i: (0, 0, 0)),
            pl.BlockSpec((_NL, 1, _D), lambda i: (0, 0, 0)),
            pl.BlockSpec((_NL, 1, _D), lambda i: (0, 0, 0)),
        ],
        out_specs=pl.BlockSpec((_PM, _D), lambda i: (i, 0)),
        out_shape=jax.ShapeDtypeStruct((n, _D), jnp.float32),
        compiler_params=pltpu.CompilerParams(
            dimension_semantics=("arbitrary",)),
    )(af2, d2, w1, b1, w2, b2, gw, gb)


def kernel(x, y, y_atomtypes, params, x_batch, y_batch):
    n = x.shape[0]

    xb_blk = x_batch.reshape(n // _P, _P)
    blo = xb_blk[:, 0]
    bhi = xb_blk[:, _P - 1]
    wlo = jnp.searchsorted(y_batch, blo, side="left").astype(jnp.int32)
    whi = jnp.searchsorted(y_batch, bhi, side="right").astype(jnp.int32)
    t0 = wlo // _TA
    t1 = (whi + _TA - 1) // _TA

    idx, d2 = _knn_call(
        x,
        y.T,
        x_batch.reshape(n, 1),
        y_batch.reshape(1, y.shape[0]),
        t0,
        t1,
    )

    af = _gather_call(y_atomtypes, idx.reshape(-1))
    af2 = af.reshape(n, _K * _D)

    w1 = jnp.stack(params["w1"])
    b1 = jnp.stack(params["b1"]).reshape(_NL, 1, _H)
    w2 = jnp.stack(params["w2"])
    b2 = jnp.stack(params["b2"]).reshape(_NL, 1, _D)
    gw = jnp.stack(params["gw"]).reshape(_NL, 1, _D)
    gb = jnp.stack(params["gb"]).reshape(_NL, 1, _D)

    return _mlp_call(af2, d2, w1, b1, w2, b2, gw, gb)


# speedup vs baseline: 1.1131x; 1.0635x over previous
import functools

import jax
import jax.numpy as jnp
from jax import lax
from jax.experimental import pallas as pl
from jax.experimental.pallas import tpu as pltpu
from jax.experimental.pallas import tpu_sc as plsc

_D = 16
_K = 16
_NL = 3
_H = 2 * _D + 1

_P = 512
_TA = 1024

_NC = 2
_NS = 16
_NW = _NC * _NS
_R = 128
_CR = 8


def _knn_body(t0_ref, t1_ref, x_ref, xb_ref, yt_ref, yb_ref, idx_ref, d2_ref,
              dscr):
    i = pl.program_id(0)
    t0 = t0_ref[i]
    t1 = t1_ref[i]
    xx = x_ref[:, 0:1]
    xy = x_ref[:, 1:2]
    xz = x_ref[:, 2:3]
    xb = xb_ref[:, 0:1]

    inf = jnp.float32(jnp.inf)
    big = jnp.int32(2**30)
    lane = lax.broadcasted_iota(jnp.int32, (_P, _TA), 1)

    nex = 2

    def topn_of_tile(dt, c0):
        out = []
        cur = dt
        for j in range(nex):
            m = jnp.min(cur, axis=1, keepdims=True)
            l = jnp.min(jnp.where(cur == m, lane, big), axis=1, keepdims=True)
            out.append((m, l + c0))
            if j + 1 < nex:
                cur = jnp.where(lane == l, inf, cur)
        return out

    def cmpx(a, b):
        c = a[0] <= b[0]
        lo = (jnp.where(c, a[0], b[0]), jnp.where(c, a[1], b[1]))
        hi = (jnp.where(c, b[0], a[0]), jnp.where(c, b[1], a[1]))
        return lo, hi

    def mergen(a, b):
        c = [None] * nex
        for j in range(nex):
            av, bv = a[j], b[nex - 1 - j]
            keep = av[0] <= bv[0]
            c[j] = (jnp.where(keep, av[0], bv[0]),
                    jnp.where(keep, av[1], bv[1]))
        if nex == 4:
            c[0], c[2] = cmpx(c[0], c[2])
            c[1], c[3] = cmpx(c[1], c[3])
            c[0], c[1] = cmpx(c[0], c[1])
            c[2], c[3] = cmpx(c[2], c[3])
        else:
            c[0], c[1] = cmpx(c[0], c[1])
        return c

    def carry0():
        z = jnp.full((_P, 1), inf, jnp.float32)
        zi = jnp.zeros((_P, 1), jnp.int32)
        return [(z, zi)] * nex

    def pass0_body(t, carry):
        c0 = t * _TA
        dx = xx - yt_ref[0:1, pl.ds(c0, _TA)]
        dy = xy - yt_ref[1:2, pl.ds(c0, _TA)]
        dz = xz - yt_ref[2:3, pl.ds(c0, _TA)]
        dt = dx * dx + dy * dy + dz * dz
        dt = jnp.where(xb != yb_ref[0:1, pl.ds(c0, _TA)], inf, dt)
        dscr[:, pl.ds(c0, _TA)] = dt
        return mergen(carry, topn_of_tile(dt, c0))

    sel = lax.fori_loop(t0, t1, pass0_body, carry0())
    picks = list(sel)

    for _ in range(_K // nex - 1):
        vlast = picks[-1][0]

        def scan_body(t, carry, vlast=vlast):
            c0 = t * _TA
            dt = dscr[:, pl.ds(c0, _TA)]
            dt = jnp.where(dt <= vlast, inf, dt)
            return mergen(carry, topn_of_tile(dt, c0))

        sel = lax.fori_loop(t0, t1, scan_body, carry0())
        picks += list(sel)

    idx_ref[:, :] = jnp.concatenate([p[1] for p in picks], axis=1)
    d2_ref[:, :] = jnp.concatenate([p[0] for p in picks], axis=1)


def _knn_call(x, yt, xb2, yb2, t0, t1):
    n = x.shape[0]
    v = yt.shape[1]
    nb = n // _P
    return pl.pallas_call(
        _knn_body,
        grid=(nb,),
        in_specs=[
            pl.BlockSpec(memory_space=pltpu.SMEM),
            pl.BlockSpec(memory_space=pltpu.SMEM),
            pl.BlockSpec((_P, 3), lambda i: (i, 0)),
            pl.BlockSpec((_P, 1), lambda i: (i, 0)),
            pl.BlockSpec((3, v), lambda i: (0, 0)),
            pl.BlockSpec((1, v), lambda i: (0, 0)),
        ],
        out_specs=[
            pl.BlockSpec((_P, _K), lambda i: (i, 0)),
            pl.BlockSpec((_P, _K), lambda i: (i, 0)),
        ],
        out_shape=[
            jax.ShapeDtypeStruct((n, _K), jnp.int32),
            jax.ShapeDtypeStruct((n, _K), jnp.float32),
        ],
        scratch_shapes=[pltpu.VMEM((_P, v), jnp.float32)],
        compiler_params=pltpu.CompilerParams(
            dimension_semantics=("arbitrary",)),
    )(t0, t1, x, xb2, yt, yb2)


def _gather_body(tab_hbm, idx_hbm, out_hbm, idx_v, buf_v, sem):
    wid = lax.axis_index("s") * _NC + lax.axis_index("c")
    rows_per_w = idx_hbm.shape[0] // _NW
    base = wid * rows_per_w
    pltpu.sync_copy(idx_hbm.at[pl.ds(base, rows_per_w)], idx_v)

    def chunk(ci, carry):
        handles = []
        for j in range(_CR):
            r = ci * _CR + j
            h = pltpu.async_copy(
                tab_hbm.at[idx_v.at[r]],
                buf_v.at[pl.ds(j * _R, _R)],
                sem,
            )
            handles.append(h)
        for h in handles:
            h.wait()
        out_off = (base + ci * _CR) * _R
        pltpu.sync_copy(buf_v, out_hbm.at[pl.ds(out_off, _CR * _R)])
        return carry

    lax.fori_loop(0, rows_per_w // _CR, chunk, 0)


def _gather_call(table, idx_flat):
    b = idx_flat.shape[0]
    d = table.shape[1]
    idx2 = idx_flat.reshape(b // _R, _R)
    mesh = plsc.VectorSubcoreMesh(core_axis_name="c", subcore_axis_name="s")
    rows_per_w = idx2.shape[0] // _NW
    run = functools.partial(
        pl.kernel,
        mesh=mesh,
        out_type=jax.ShapeDtypeStruct((b, d), jnp.float32),
        scratch_types=[
            pltpu.VMEM((rows_per_w, _R), jnp.int32),
            pltpu.VMEM((_CR * _R, d), jnp.float32),
            pltpu.SemaphoreType.DMA,
        ],
        compiler_params=pltpu.CompilerParams(use_tc_tiling_on_sc=False),
    )(_gather_body)
    return run(table, idx2)


_PM = 1024


def _mlp_body(af_ref, dt_ref, w1_ref, b1_ref, w2_ref, b2_ref, gw_ref, gb_ref,
              out_ref):
    pe = jnp.ones((_PM, _D), jnp.float32)
    for l in range(_NL):
        w1 = w1_ref[l]
        w1a = w1[0:_D, :]
        w1b = w1[_D:2 * _D, :]
        w1c = w1[2 * _D:2 * _D + 1, :]
        peh = jnp.dot(pe, w1a, preferred_element_type=jnp.float32) + b1_ref[l]
        hsum = jnp.zeros((_PM, _H), jnp.float32)
        for k in range(_K):
            af = af_ref[:, k * _D:(k + 1) * _D]
            dk = dt_ref[:, k:k + 1]
            hk = (peh + jnp.dot(af, w1b, preferred_element_type=jnp.float32)
                  + dk * w1c)
            hsum = hsum + jnp.where(hk >= 0, hk, 0.2 * hk)
        msg = (jnp.dot(hsum, w2_ref[l], preferred_element_type=jnp.float32)
               + jnp.float32(_K) * b2_ref[l])
        g1 = msg[:, 0:_D // 2]
        g2 = msg[:, _D // 2:_D]
        mu1 = jnp.mean(g1, axis=1, keepdims=True)
        mu2 = jnp.mean(g2, axis=1, keepdims=True)
        c1 = g1 - mu1
        c2 = g2 - mu2
        v1 = jnp.mean(c1 * c1, axis=1, keepdims=True)
        v2 = jnp.mean(c2 * c2, axis=1, keepdims=True)
        tn = jnp.concatenate(
            [c1 / jnp.sqrt(v1 + 1e-5), c2 / jnp.sqrt(v2 + 1e-5)], axis=1)
        tn = tn * gw_ref[l] + gb_ref[l]
        pe = pe + jnp.where(tn >= 0, tn, 0.2 * tn)
    out_ref[:, :] = pe


def _mlp_call(af2, d2, w1, b1, w2, b2, gw, gb):
    n = af2.shape[0]
    return pl.pallas_call(
        _mlp_body,
        grid=(n // _PM,),
        in_specs=[
            pl.BlockSpec((_PM, _K * _D), lambda i: (i, 0)),
            pl.BlockSpec((_PM, _K), lambda i: (i, 0)),
            pl.BlockSpec((_NL, _H, _H), lambda i: (0, 0, 0)),
            pl.BlockSpec((_NL, 1, _H), lambda i: (0, 0, 0)),
            pl.BlockSpec((_NL, _H, _D), lambda i: (0, 0, 0)),
            pl.BlockSpec((_NL, 1, _D), lambda i: (0, 0, 0)),
            pl.BlockSpec((_NL, 1, _D), lambda i: (0, 0, 0)),
            pl.BlockSpec((_NL, 1, _D), lambda i: (0, 0, 0)),
        ],
        out_specs=pl.BlockSpec((_PM, _D), lambda i: (i, 0)),
        out_shape=jax.ShapeDtypeStruct((n, _D), jnp.float32),
        compiler_params=pltpu.CompilerParams(
            dimension_semantics=("arbitrary",)),
    )(af2, d2, w1, b1, w2, b2, gw, gb)


def kernel(x, y, y_atomtypes, params, x_batch, y_batch):
    n = x.shape[0]

    xb_blk = x_batch.reshape(n // _P, _P)
    blo = xb_blk[:, 0]
    bhi = xb_blk[:, _P - 1]
    wlo = jnp.searchsorted(y_batch, blo, side="left").astype(jnp.int32)
    whi = jnp.searchsorted(y_batch, bhi, side="right").astype(jnp.int32)
    t0 = wlo // _TA
    t1 = (whi + _TA - 1) // _TA

    idx, d2 = _knn_call(
        x,
        y.T,
        x_batch.reshape(n, 1),
        y_batch.reshape(1, y.shape[0]),
        t0,
        t1,
    )

    af = _gather_call(y_atomtypes, idx.reshape(-1))
    af2 = af.reshape(n, _K * _D)

    w1 = jnp.stack(params["w1"])
    b1 = jnp.stack(params["b1"]).reshape(_NL, 1, _H)
    w2 = jnp.stack(params["w2"])
    b2 = jnp.stack(params["b2"]).reshape(_NL, 1, _D)
    gw = jnp.stack(params["gw"]).reshape(_NL, 1, _D)
    gb = jnp.stack(params["gb"]).reshape(_NL, 1, _D)

    return _mlp_call(af2, d2, w1, b1, w2, b2, gw, gb)
